# baseline (speedup 1.0000x reference)
import jax
import jax.numpy as jnp
from jax.experimental import pallas as pl


def _mm_kernel(a_ref, x_ref, o_ref):
    a = a_ref[...].astype(jnp.bfloat16)
    o_ref[...] = jnp.dot(a, x_ref[...], preferred_element_type=jnp.float32)


def kernel(adj, embeds):
    m, kdim = adj.shape
    _, d = embeds.shape
    bm = 128
    x16 = embeds.astype(jnp.bfloat16)
    return pl.pallas_call(
        _mm_kernel,
        grid=(pl.cdiv(m, bm),),
        in_specs=[
            pl.BlockSpec((bm, kdim), lambda i: (i, 0)),
            pl.BlockSpec((kdim, d), lambda i: (0, 0)),
        ],
        out_specs=pl.BlockSpec((bm, d), lambda i: (i, 0)),
        out_shape=jax.ShapeDtypeStruct((m, d), jnp.float32),
    )(adj, x16)


# speedup vs baseline: 1.1794x; 1.1794x over previous
import jax
import jax.numpy as jnp
from jax import lax
from jax.experimental import pallas as pl


def _mm_kernel(a_ref, x_ref, o_ref):
    o_ref[...] = jnp.dot(
        a_ref[...],
        x_ref[...],
        preferred_element_type=jnp.float32,
        precision=lax.Precision.DEFAULT,
    )


def kernel(adj, embeds):
    m, kdim = adj.shape
    _, d = embeds.shape
    bm = 256
    return pl.pallas_call(
        _mm_kernel,
        grid=(pl.cdiv(m, bm),),
        in_specs=[
            pl.BlockSpec((bm, kdim), lambda i: (i, 0)),
            pl.BlockSpec((kdim, d), lambda i: (0, 0)),
        ],
        out_specs=pl.BlockSpec((bm, d), lambda i: (i, 0)),
        out_shape=jax.ShapeDtypeStruct((m, d), jnp.float32),
    )(adj, embeds)
